# baseline (speedup 1.0000x reference)
import functools

import jax
import jax.numpy as jnp
from jax import lax
from jax.experimental import pallas as pl
from jax.experimental.pallas import tpu as pltpu
from jax.experimental.pallas import tpu_sc as plsc

VOCAB = 100000
EMBED_DIM = 64
BATCH = 4096
HIST_LEN = 50

NUM_CORES = 2
NUM_SUBCORES = 16
NUM_WORKERS = NUM_CORES * NUM_SUBCORES
BPW = BATCH // NUM_WORKERS
CHUNK = BPW
NSTREAM = HIST_LEN
NBUF = 5
NWAVES = NSTREAM // NBUF
UNROLL = 8
NLANE = 16
NVEC = EMBED_DIM // NLANE


def _body(idx_hbm, table_hbm, out_hbm, idx_v, b0, b1, b2, b3, b4,
          acc_v, s0, s1, s2, s3, s4):
    ring = (b0, b1, b2, b3, b4)
    sems = (s0, s1, s2, s3, s4)
    wid = lax.axis_index("s") * NUM_CORES + lax.axis_index("c")

    pltpu.sync_copy(idx_hbm.at[wid], idx_v)

    zero = jnp.zeros((NLANE,), jnp.float32)

    def zbody(b, carry):
        for j in range(NVEC):
            acc_v[b, pl.ds(j * NLANE, NLANE)] = zero
        return carry

    lax.fori_loop(0, BPW, zbody, None)

    def reduce_one(s, k):
        pltpu.make_async_copy(
            table_hbm.at[idx_v.at[s]], ring[k], sems[k]).wait()
        buf = ring[k]

        @plsc.parallel_loop(0, CHUNK, unroll=UNROLL)
        def rbody(i):
            for j in range(NVEC):
                acc_v[i, pl.ds(j * NLANE, NLANE)] += (
                    buf[i, pl.ds(j * NLANE, NLANE)])

    for k in range(NBUF):
        pltpu.async_copy(table_hbm.at[idx_v.at[k]], ring[k], sems[k])

    def wbody(w, carry):
        for k in range(NBUF):
            s = w * NBUF + k
            reduce_one(s, k)
            pltpu.async_copy(
                table_hbm.at[idx_v.at[s + NBUF]], ring[k], sems[k])
        return carry

    lax.fori_loop(0, NWAVES - 1, wbody, None)

    for k in range(NBUF):
        reduce_one((NWAVES - 1) * NBUF + k, k)

    scale = jnp.float32(1.0 / HIST_LEN)

    def sbody(b, carry):
        for j in range(NVEC):
            acc_v[b, pl.ds(j * NLANE, NLANE)] = (
                acc_v[b, pl.ds(j * NLANE, NLANE)] * scale)
        return carry

    lax.fori_loop(0, BPW, sbody, None)
    pltpu.sync_copy(acc_v, out_hbm.at[pl.ds(wid * BPW, BPW)])


_emb = functools.partial(
    pl.kernel,
    out_type=jax.ShapeDtypeStruct((BATCH, EMBED_DIM), jnp.float32),
    mesh=plsc.VectorSubcoreMesh(core_axis_name="c", subcore_axis_name="s"),
    compiler_params=pltpu.CompilerParams(
        use_tc_tiling_on_sc=False, needs_layout_passes=False),
    scratch_types=(
        [pltpu.VMEM((NSTREAM, CHUNK), jnp.int32)]
        + [pltpu.VMEM((CHUNK, EMBED_DIM), jnp.float32)] * NBUF
        + [pltpu.VMEM((BPW, EMBED_DIM), jnp.float32)]
        + [pltpu.SemaphoreType.DMA] * NBUF
    ),
)(_body)


def kernel(event_type, table):
    idx = (event_type.astype(jnp.int32)
           .reshape(NUM_WORKERS, BPW, HIST_LEN)
           .transpose(0, 2, 1))
    return _emb(idx, table).reshape(BATCH, 1, EMBED_DIM)


# speedup vs baseline: 1.0052x; 1.0052x over previous
import functools

import jax
import jax.numpy as jnp
from jax import lax
from jax.experimental import pallas as pl
from jax.experimental.pallas import tpu as pltpu
from jax.experimental.pallas import tpu_sc as plsc

VOCAB = 100000
EMBED_DIM = 64
BATCH = 4096
HIST_LEN = 50

NUM_CORES = 2
NUM_SUBCORES = 16
NUM_WORKERS = NUM_CORES * NUM_SUBCORES
BPW = BATCH // NUM_WORKERS
CHUNK = BPW
NSTREAM = HIST_LEN
NBUF = 5
NWAVES = NSTREAM // NBUF
UNROLL = 8
NLANE = 16
NVEC = EMBED_DIM // NLANE


def _body(idx_hbm, table_hbm, out_hbm, idx_v, b0, b1, b2, b3, b4,
          acc_v, s0, s1, s2, s3, s4):
    ring = (b0, b1, b2, b3, b4)
    sems = (s0, s1, s2, s3, s4)
    wid = lax.axis_index("s") * NUM_CORES + lax.axis_index("c")

    pltpu.sync_copy(idx_hbm.at[wid], idx_v)

    def reduce_one(s, k):
        pltpu.make_async_copy(
            table_hbm.at[idx_v.at[s]], ring[k], sems[k]).wait()
        buf = ring[k]

        @plsc.parallel_loop(0, CHUNK, unroll=UNROLL)
        def rbody(i):
            for j in range(NVEC):
                acc_v[i, pl.ds(j * NLANE, NLANE)] += (
                    buf[i, pl.ds(j * NLANE, NLANE)])

    for k in range(NBUF):
        pltpu.async_copy(table_hbm.at[idx_v.at[k]], ring[k], sems[k])

    zero = jnp.zeros((NLANE,), jnp.float32)

    @plsc.parallel_loop(0, BPW, unroll=UNROLL)
    def zbody(b):
        for j in range(NVEC):
            acc_v[b, pl.ds(j * NLANE, NLANE)] = zero

    def wbody(w, carry):
        for k in range(NBUF):
            s = w * NBUF + k
            reduce_one(s, k)
            pltpu.async_copy(
                table_hbm.at[idx_v.at[s + NBUF]], ring[k], sems[k])
        return carry

    lax.fori_loop(0, NWAVES - 1, wbody, None)

    for k in range(NBUF):
        reduce_one((NWAVES - 1) * NBUF + k, k)

    scale = jnp.float32(1.0 / HIST_LEN)

    @plsc.parallel_loop(0, BPW, unroll=UNROLL)
    def sbody(b):
        for j in range(NVEC):
            acc_v[b, pl.ds(j * NLANE, NLANE)] = (
                acc_v[b, pl.ds(j * NLANE, NLANE)] * scale)

    pltpu.sync_copy(acc_v, out_hbm.at[pl.ds(wid * BPW, BPW)])


_emb = functools.partial(
    pl.kernel,
    out_type=jax.ShapeDtypeStruct((BATCH, EMBED_DIM), jnp.float32),
    mesh=plsc.VectorSubcoreMesh(core_axis_name="c", subcore_axis_name="s"),
    compiler_params=pltpu.CompilerParams(
        use_tc_tiling_on_sc=False, needs_layout_passes=False),
    scratch_types=(
        [pltpu.VMEM((NSTREAM, CHUNK), jnp.int32)]
        + [pltpu.VMEM((CHUNK, EMBED_DIM), jnp.float32)] * NBUF
        + [pltpu.VMEM((BPW, EMBED_DIM), jnp.float32)]
        + [pltpu.SemaphoreType.DMA] * NBUF
    ),
)(_body)


def kernel(event_type, table):
    idx = (event_type.astype(jnp.int32)
           .reshape(NUM_WORKERS, BPW, HIST_LEN)
           .transpose(0, 2, 1))
    return _emb(idx, table).reshape(BATCH, 1, EMBED_DIM)


# speedup vs baseline: 1.0070x; 1.0018x over previous
import functools

import jax
import jax.numpy as jnp
from jax import lax
from jax.experimental import pallas as pl
from jax.experimental.pallas import tpu as pltpu
from jax.experimental.pallas import tpu_sc as plsc

VOCAB = 100000
EMBED_DIM = 64
BATCH = 4096
HIST_LEN = 50

NUM_CORES = 2
NUM_SUBCORES = 16
NUM_WORKERS = NUM_CORES * NUM_SUBCORES
BPW = BATCH // NUM_WORKERS
CHUNK = BPW
NSTREAM = HIST_LEN
NBUF = 5
NWAVES = NSTREAM // NBUF
UNROLL = 8
NLANE = 16
NVEC = EMBED_DIM // NLANE


def _body(idx_hbm, table_hbm, out_hbm, idx_v, b0, b1, b2, b3, b4,
          acc_v, s0, s1, s2, s3, s4, sem_idx):
    ring = (b0, b1, b2, b3, b4)
    sems = (s0, s1, s2, s3, s4)
    wid = lax.axis_index("s") * NUM_CORES + lax.axis_index("c")

    pltpu.sync_copy(idx_hbm.at[wid, pl.ds(0, NBUF)],
                    idx_v.at[pl.ds(0, NBUF)])
    rest_cp = pltpu.async_copy(
        idx_hbm.at[wid, pl.ds(NBUF, NSTREAM - NBUF)],
        idx_v.at[pl.ds(NBUF, NSTREAM - NBUF)], sem_idx)

    def reduce_one(s, k):
        pltpu.make_async_copy(
            table_hbm.at[idx_v.at[s]], ring[k], sems[k]).wait()
        buf = ring[k]

        @plsc.parallel_loop(0, CHUNK, unroll=UNROLL)
        def rbody(i):
            for j in range(NVEC):
                acc_v[i, pl.ds(j * NLANE, NLANE)] += (
                    buf[i, pl.ds(j * NLANE, NLANE)])

    for k in range(NBUF):
        pltpu.async_copy(table_hbm.at[idx_v.at[k]], ring[k], sems[k])

    zero = jnp.zeros((NLANE,), jnp.float32)

    @plsc.parallel_loop(0, BPW, unroll=UNROLL)
    def zbody(b):
        for j in range(NVEC):
            acc_v[b, pl.ds(j * NLANE, NLANE)] = zero

    rest_cp.wait()

    def wbody(w, carry):
        for k in range(NBUF):
            s = w * NBUF + k
            reduce_one(s, k)
            pltpu.async_copy(
                table_hbm.at[idx_v.at[s + NBUF]], ring[k], sems[k])
        return carry

    lax.fori_loop(0, NWAVES - 1, wbody, None)

    for k in range(NBUF):
        reduce_one((NWAVES - 1) * NBUF + k, k)

    scale = jnp.float32(1.0 / HIST_LEN)

    @plsc.parallel_loop(0, BPW, unroll=UNROLL)
    def sbody(b):
        for j in range(NVEC):
            acc_v[b, pl.ds(j * NLANE, NLANE)] = (
                acc_v[b, pl.ds(j * NLANE, NLANE)] * scale)

    pltpu.sync_copy(acc_v, out_hbm.at[pl.ds(wid * BPW, BPW)])


_emb = functools.partial(
    pl.kernel,
    out_type=jax.ShapeDtypeStruct((BATCH, EMBED_DIM), jnp.float32),
    mesh=plsc.VectorSubcoreMesh(core_axis_name="c", subcore_axis_name="s"),
    compiler_params=pltpu.CompilerParams(
        use_tc_tiling_on_sc=False, needs_layout_passes=False),
    scratch_types=(
        [pltpu.VMEM((NSTREAM, CHUNK), jnp.int32)]
        + [pltpu.VMEM((CHUNK, EMBED_DIM), jnp.float32)] * NBUF
        + [pltpu.VMEM((BPW, EMBED_DIM), jnp.float32)]
        + [pltpu.SemaphoreType.DMA] * (NBUF + 1)
    ),
)(_body)


def kernel(event_type, table):
    idx = (event_type.astype(jnp.int32)
           .reshape(NUM_WORKERS, BPW, HIST_LEN)
           .transpose(0, 2, 1))
    return _emb(idx, table).reshape(BATCH, 1, EMBED_DIM)
